# Initial kernel scaffold; baseline (speedup 1.0000x reference)
import jax
import jax.numpy as jnp
from jax.experimental import pallas as pl


def kernel(n_features, c_features, emb1, emb2, W_dense, b_dense, W1, b1, W2, b2, Wo, bo):
    raise NotImplementedError("write your pallas kernel here")


# speedup vs baseline: 3.4171x; 3.4171x over previous
import functools

import jax
import jax.numpy as jnp
import numpy as np
from jax import lax
from jax.experimental import pallas as pl
from jax.experimental.pallas import tpu as pltpu
from jax.experimental.pallas import tpu_sc as plsc

B, G = 4096, 20
FEA_N = 13
FEA_C = 26
V = 100000
D = 16
R = B * G
RT = R * FEA_C
CH = 128
NCHUNK = RT // CH
NW = 32
PWC = NCHUNK // NW
SUP = 8
NSUP = PWC // SUP

BLK = 512


def _sc_gather(idx2d, e2_tab, e1_tab):
    mesh = plsc.VectorSubcoreMesh(core_axis_name="c", subcore_axis_name="s")

    @functools.partial(
        pl.kernel,
        mesh=mesh,
        out_type=[
            jax.ShapeDtypeStruct((NCHUNK, CH, D), jnp.float32),
            jax.ShapeDtypeStruct((NCHUNK, CH), jnp.float32),
        ],
        scratch_types=[
            pltpu.VMEM((SUP, CH), jnp.int32),
            pltpu.VMEM((SUP, CH, D), jnp.float32),
            pltpu.VMEM((SUP, CH), jnp.float32),
            pltpu.SemaphoreType.DMA,
            pltpu.SemaphoreType.DMA,
        ],
        compiler_params=pltpu.CompilerParams(use_tc_tiling_on_sc=False),
    )
    def k(idx_hbm, e2_hbm, e1_hbm, oute2, oute1, idx_v, e2_v, e1_v, sem2, sem1):
        wid = lax.axis_index("s") * 2 + lax.axis_index("c")
        base = wid * PWC

        def body(s, carry):
            row0 = base + s * SUP
            pltpu.sync_copy(idx_hbm.at[pl.ds(row0, SUP)], idx_v)
            cps = []
            for j in range(SUP):
                c2 = pltpu.async_copy(e2_hbm.at[idx_v.at[j]], e2_v.at[j], sem2)
                c1 = pltpu.async_copy(e1_hbm.at[idx_v.at[j]], e1_v.at[j], sem1)
                cps.append((c2, c1))
            for c2, c1 in cps:
                c2.wait()
                c1.wait()
            pltpu.sync_copy(e2_v, oute2.at[pl.ds(row0, SUP)])
            pltpu.sync_copy(e1_v, oute1.at[pl.ds(row0, SUP)])
            return carry

        lax.fori_loop(0, NSUP, body, 0)

    return k(idx2d, e2_tab, e1_tab)


def _tc_body(e2_ref, e1_ref, nf_ref, S_ref, W1a_ref, W1b_ref, Wd_ref,
             W2_ref, WoD_ref, wo12_ref, b1_ref, b2_ref, bo_ref, bd_ref,
             out_ref):
    f32 = jnp.float32
    e2 = e2_ref[...]
    nf = nf_ref[...]
    fm1 = (jnp.sum(e1_ref[...], axis=1, keepdims=True)
           + jnp.dot(nf, Wd_ref[...], preferred_element_type=f32)
           + bd_ref[0, 0])
    S = S_ref[...]
    se = jnp.dot(e2, S, preferred_element_type=f32)
    ssq = jnp.dot(e2 * e2, S, preferred_element_type=f32)
    fm2 = 0.5 * jnp.sum(se * se - ssq, axis=1, keepdims=True)
    h = jnp.dot(e2, W1a_ref[...], preferred_element_type=f32)
    h += jnp.dot(nf, W1b_ref[...], preferred_element_type=f32)
    h = jnp.maximum(h + b1_ref[...], 0.0)
    h2 = jnp.dot(h, W2_ref[...], preferred_element_type=f32)
    h2 = jnp.maximum(h2 + b2_ref[...], 0.0)
    wo12 = wo12_ref[...]
    out = jnp.dot(h2, WoD_ref[...], preferred_element_type=f32)
    out += fm1 * wo12[0:1, :] + fm2 * wo12[1:2, :] + bo_ref[...]
    out_ref[...] = jnp.maximum(out, 0.0)


def _tc_fused(e2f, e1m, nf, S, W1a, W1b, Wd, W2, WoD, wo12, b1, b2, bo, bd):
    nblk = R // BLK
    row = lambda i: (i, 0)
    full = lambda i: (0, 0)
    return pl.pallas_call(
        _tc_body,
        grid=(nblk,),
        in_specs=[
            pl.BlockSpec((BLK, FEA_C * D), row),
            pl.BlockSpec((BLK, FEA_C), row),
            pl.BlockSpec((BLK, FEA_N), row),
            pl.BlockSpec((FEA_C * D, D), full),
            pl.BlockSpec((FEA_C * D, 256), full),
            pl.BlockSpec((FEA_N, 256), full),
            pl.BlockSpec((FEA_N, 1), full),
            pl.BlockSpec((256, 128), full),
            pl.BlockSpec((128, 128), full),
            pl.BlockSpec((2, 128), full),
            pl.BlockSpec((1, 256), full),
            pl.BlockSpec((1, 128), full),
            pl.BlockSpec((1, 128), full),
            pl.BlockSpec((1, 1), full),
        ],
        out_specs=pl.BlockSpec((BLK, 128), row),
        out_shape=jax.ShapeDtypeStruct((R, 128), jnp.float32),
    )(e2f, e1m, nf, S, W1a, W1b, Wd, W2, WoD, wo12, b1, b2, bo, bd)


def kernel(n_features, c_features, emb1, emb2, W_dense, b_dense,
           W1, b1, W2, b2, Wo, bo):
    cf = c_features.astype(jnp.int32)
    offs = (jnp.arange(FEA_C, dtype=jnp.int32) * V)
    idx2d = (cf + offs).reshape(NCHUNK, CH)
    e2_tab = emb2.reshape(FEA_C * V, D)
    e1_tab = emb1.reshape(FEA_C * V)

    ge2, ge1 = _sc_gather(idx2d, e2_tab, e1_tab)
    e2f = ge2.reshape(R, FEA_C * D)
    e1m = ge1.reshape(R, FEA_C)
    nf = n_features.reshape(R, FEA_N)

    S = jnp.asarray(np.tile(np.eye(D, dtype=np.float32), (FEA_C, 1)))
    W1a = W1[:FEA_C * D]
    W1b = W1[FEA_C * D:]
    wo12 = Wo[:2]
    WoD = Wo[2:]

    out = _tc_fused(e2f, e1m, nf, S, W1a, W1b, Wd=W_dense, W2=W2, WoD=WoD,
                    wo12=wo12, b1=b1.reshape(1, 256), b2=b2.reshape(1, 128),
                    bo=bo.reshape(1, 128), bd=b_dense.reshape(1, 1))
    return out.reshape(B, G, 128)
